# baseline (speedup 1.0000x reference)
import functools

import jax
import jax.numpy as jnp
from jax import lax
from jax.experimental import pallas as pl
from jax.experimental.pallas import tpu as pltpu
from jax.experimental.pallas import tpu_sc as plsc

N = 10000
E = 320000
D = 128

NC = 2
NS = 16
NW = NC * NS
EPW = E // NW
CH = 128
NFULL = EPW // CH
REM = EPW - NFULL * CH
CHA = 112
NFA = EPW // CHA
REMA = EPW - NFA * CHA
RPT = 624
RTAIL = N - NS * RPT

_mesh = plsc.VectorSubcoreMesh(core_axis_name="c", subcore_axis_name="s")


def _zero_spmem(zeros_hbm, acc, s):
    r0 = s * RPT
    full, rem = RPT // CH, RPT % CH
    for k in range(full):
        pltpu.sync_copy(zeros_hbm, acc.at[pl.ds(r0 + k * CH, CH)])
    if rem:
        pltpu.sync_copy(zeros_hbm.at[pl.ds(0, rem)],
                        acc.at[pl.ds(r0 + full * CH, rem)])

    @pl.when(s == NS - 1)
    def _():
        pltpu.sync_copy(zeros_hbm.at[pl.ds(0, RTAIL)],
                        acc.at[pl.ds(NS * RPT, RTAIL)])


def _flush_spmem(acc, out_hbm, c, s):
    r0 = s * RPT
    pltpu.sync_copy(acc.at[pl.ds(r0, RPT)], out_hbm.at[c, pl.ds(r0, RPT)])

    @pl.when(s == NS - 1)
    def _():
        pltpu.sync_copy(acc.at[pl.ds(NS * RPT, RTAIL)],
                        out_hbm.at[c, pl.ds(NS * RPT, RTAIL)])


@functools.partial(
    pl.kernel,
    out_type=jax.ShapeDtypeStruct((NC, N, D), jnp.float32),
    mesh=_mesh,
    scratch_types=[
        pltpu.VMEM((4, CH), jnp.int32),
        pltpu.VMEM((REM,), jnp.int32),
        pltpu.VMEM((CH, D), jnp.float32),
        pltpu.VMEM_SHARED((N, D), jnp.float32),
        [pltpu.SemaphoreType.DMA] * 4,
        [pltpu.SemaphoreType.DMA] * 2,
    ],
)
def _sc_degree(dst_hbm, ones_hbm, zeros_hbm, out_hbm, di, dr, ones_v, acc,
               dsem, ssem):
    c = lax.axis_index("c")
    s = lax.axis_index("s")
    ebase = (c * NS + s) * EPW

    _zero_spmem(zeros_hbm, acc, s)
    pltpu.sync_copy(ones_hbm, ones_v)
    plsc.subcore_barrier()

    def load_idx(jj, b):
        pltpu.async_copy(dst_hbm.at[pl.ds(ebase + jj * CH, CH)], di.at[b],
                         dsem[b])

    def wait_idx(b):
        pltpu.make_async_copy(dst_hbm.at[pl.ds(ebase, CH)], di.at[b],
                              dsem[b]).wait()

    def start_scatter(b, p):
        pltpu.async_copy(ones_v, acc.at[di.at[b]], ssem[p], add=True)

    def wait_scatter(b, p):
        pltpu.make_async_copy(ones_v, acc.at[di.at[b]], ssem[p]).wait()

    for b in range(4):
        load_idx(b, b)
    wait_idx(0)
    start_scatter(0, 0)
    wait_idx(1)
    start_scatter(1, 1)

    def body(i, carry):
        k0 = 2 + 4 * i
        for m in range(4):
            k, sk, sk2, p = k0 + m, (2 + m) % 4, m % 4, m % 2
            wait_scatter(sk2, p)
            load_idx(k + 2, sk2)
            wait_idx(sk)
            start_scatter(sk, p)
        return carry

    lax.fori_loop(0, (NFULL - 6) // 4, body, 0)

    for m in range(2):
        k, sk, sk2, p = NFULL - 4 + m, (2 + m) % 4, m % 4, m % 2
        wait_scatter(sk2, p)
        load_idx(k + 2, sk2)
        wait_idx(sk)
        start_scatter(sk, p)
    for m in range(2):
        sk, p = m % 4, m % 2
        wait_scatter((2 + m) % 4, p)
        wait_idx(sk)
        start_scatter(sk, p)
    wait_scatter(0, 0)
    wait_scatter(1, 1)

    eb = ebase + NFULL * CH
    pltpu.sync_copy(dst_hbm.at[pl.ds(eb, REM)], dr)
    pltpu.sync_copy(ones_v.at[pl.ds(0, REM)], acc.at[dr], add=True)
    plsc.subcore_barrier()

    _flush_spmem(acc, out_hbm, c, s)


@functools.partial(
    pl.kernel,
    out_type=jax.ShapeDtypeStruct((NC, N, D), jnp.float32),
    mesh=_mesh,
    scratch_types=[
        pltpu.VMEM((6, CHA), jnp.int32),
        pltpu.VMEM((6, CHA), jnp.int32),
        pltpu.VMEM((REMA,), jnp.int32),
        pltpu.VMEM((REMA,), jnp.int32),
        pltpu.VMEM((3, CHA, D), jnp.float32),
        pltpu.VMEM_SHARED((N, D), jnp.float32),
        [pltpu.SemaphoreType.DMA] * 3,
        [pltpu.SemaphoreType.DMA] * 6,
        [pltpu.SemaphoreType.DMA] * 3,
    ],
)
def _sc_edge_agg(hs_hbm, src_hbm, dst_hbm, zeros_hbm, out_hbm,
                 si, di, sr, dr, rows, acc, gsem, dsem, ssem):
    c = lax.axis_index("c")
    s = lax.axis_index("s")
    ebase = (c * NS + s) * EPW

    _zero_spmem(zeros_hbm, acc, s)
    plsc.subcore_barrier()

    def load_idx(jj, b):
        eb = ebase + jj * CHA
        pltpu.async_copy(src_hbm.at[pl.ds(eb, CHA)], si.at[b], dsem[b])
        pltpu.async_copy(dst_hbm.at[pl.ds(eb, CHA)], di.at[b], dsem[b])

    def wait_idx(b):
        pltpu.make_async_copy(src_hbm.at[pl.ds(ebase, CHA)], si.at[b],
                              dsem[b]).wait()
        pltpu.make_async_copy(dst_hbm.at[pl.ds(ebase, CHA)], di.at[b],
                              dsem[b]).wait()

    def start_gather(rb, ib):
        pltpu.async_copy(hs_hbm.at[si.at[ib]], rows.at[rb], gsem[rb])

    def wait_gather(rb):
        pltpu.make_async_copy(hs_hbm.at[si.at[0]], rows.at[rb],
                              gsem[rb]).wait()

    def start_scatter(rb, ib):
        pltpu.async_copy(rows.at[rb], acc.at[di.at[ib]], ssem[rb], add=True)

    def wait_scatter(rb):
        pltpu.make_async_copy(rows.at[rb], acc.at[di.at[0]],
                              ssem[rb]).wait()

    def step(k, rb, ib, prefetch_g, prefetch_i):
        wait_gather(rb)
        start_scatter(rb, ib)
        wait_scatter((rb + 2) % 3)
        if prefetch_g:
            wait_idx((ib + 2) % 6)
            start_gather((rb + 2) % 3, (ib + 2) % 6)
        if prefetch_i:
            load_idx(k + 3, (ib + 3) % 6)

    for b in range(4):
        load_idx(b, b)
    wait_idx(0)
    start_gather(0, 0)
    wait_idx(1)
    start_gather(1, 1)
    wait_gather(0)
    start_scatter(0, 0)
    wait_idx(2)
    start_gather(2, 2)
    step(1, 1, 1, True, True)
    step(2, 2, 2, True, True)

    NLOOP = (NFA - 3 - 8) // 6

    def body(i, carry):
        for m in range(6):
            step(3 + 6 * i + m, (3 + m) % 3, (3 + m) % 6, True, True)
        return carry

    lax.fori_loop(0, NLOOP, body, 0)

    for k in range(3 + 6 * NLOOP, NFA):
        step(k, k % 3, k % 6, k + 2 <= NFA - 1, k + 3 <= NFA - 1)
    wait_scatter((NFA - 1) % 3)

    eb = ebase + NFA * CHA
    pltpu.sync_copy(src_hbm.at[pl.ds(eb, REMA)], sr)
    pltpu.sync_copy(dst_hbm.at[pl.ds(eb, REMA)], dr)
    pltpu.async_copy(hs_hbm.at[sr], rows.at[0, pl.ds(0, REMA)],
                     gsem[0]).wait()
    pltpu.sync_copy(rows.at[0, pl.ds(0, REMA)], acc.at[dr], add=True)
    plsc.subcore_barrier()

    _flush_spmem(acc, out_hbm, c, s)


BT = 1000
GRID = N // BT


def _dinv_of(degp):
    deg = degp[0, :, 0] + degp[1, :, 0] + 1.0
    return lax.rsqrt(deg)[:, None]


def _tc1_body(x_ref, w_ref, degp_ref, o_ref, od_ref):
    dinv = _dinv_of(degp_ref[...])
    o_ref[...] = jnp.dot(x_ref[...], w_ref[...],
                         preferred_element_type=jnp.float32) * dinv
    od_ref[...] = dinv


def _tc2_body(p_ref, hs_ref, dinv_ref, b_ref, w_ref, o_ref):
    dinv = dinv_ref[...]
    p = p_ref[...]
    a = jnp.maximum(dinv * (p[0] + p[1] + hs_ref[...]) + b_ref[...], 0.0)
    o_ref[...] = jnp.dot(a, w_ref[...],
                         preferred_element_type=jnp.float32) * dinv


def _tc3_body(p_ref, hs_ref, dinv_ref, b_ref, o_ref):
    dinv = dinv_ref[...]
    p = p_ref[...]
    o_ref[...] = dinv * (p[0] + p[1] + hs_ref[...]) + b_ref[...]


_row_spec = pl.BlockSpec((BT, D), lambda i: (i, 0))
_p_spec = pl.BlockSpec((NC, BT, D), lambda i: (0, i, 0))
_degp_spec = pl.BlockSpec((NC, BT, D), lambda i: (0, i, 0))
_w_spec = pl.BlockSpec((D, D), lambda i: (0, 0))
_b_spec = pl.BlockSpec((1, D), lambda i: (0, 0))
_dinv_spec = pl.BlockSpec((BT, 1), lambda i: (i, 0))
_out_nd = jax.ShapeDtypeStruct((N, D), jnp.float32)
_out_dinv = jax.ShapeDtypeStruct((N, 1), jnp.float32)

_tc1 = pl.pallas_call(
    _tc1_body, grid=(GRID,), out_shape=(_out_nd, _out_dinv),
    in_specs=[_row_spec, _w_spec, _degp_spec],
    out_specs=(_row_spec, _dinv_spec))

_tc2 = pl.pallas_call(
    _tc2_body, grid=(GRID,), out_shape=_out_nd,
    in_specs=[_p_spec, _row_spec, _dinv_spec, _b_spec, _w_spec],
    out_specs=_row_spec)

_tc3 = pl.pallas_call(
    _tc3_body, grid=(GRID,), out_shape=_out_nd,
    in_specs=[_p_spec, _row_spec, _dinv_spec, _b_spec],
    out_specs=_row_spec)


def kernel(x, edge_index, W1, b1, W2, b2):
    src = edge_index[0]
    dst = edge_index[1]
    onesD = jnp.ones((CH, D), jnp.float32)
    zerosD = jnp.zeros((CH, D), jnp.float32)

    degp = _sc_degree(dst, onesD, zerosD)
    hs1, dinv2d = _tc1(x, W1, degp)
    p1 = _sc_edge_agg(hs1, src, dst, zerosD)
    hs2 = _tc2(p1, hs1, dinv2d, b1.reshape(1, D), W2)
    p2 = _sc_edge_agg(hs2, src, dst, zerosD)
    return _tc3(p2, hs2, dinv2d, b2.reshape(1, D))


# speedup vs baseline: 1.0493x; 1.0493x over previous
import functools

import jax
import jax.numpy as jnp
from jax import lax
from jax.experimental import pallas as pl
from jax.experimental.pallas import tpu as pltpu
from jax.experimental.pallas import tpu_sc as plsc

N = 10000
E = 320000
D = 128

NC = 2
NS = 16
NW = NC * NS
EPW = E // NW
CH = 128
NFULL = EPW // CH
REM = EPW - NFULL * CH
CHA = 112
NFA = EPW // CHA
REMA = EPW - NFA * CHA
RPT = 624
RTAIL = N - NS * RPT

_mesh = plsc.VectorSubcoreMesh(core_axis_name="c", subcore_axis_name="s")


def _zero_spmem(zeros_hbm, acc, s):
    r0 = s * RPT
    full, rem = RPT // CH, RPT % CH
    for k in range(full):
        pltpu.sync_copy(zeros_hbm, acc.at[pl.ds(r0 + k * CH, CH)])
    if rem:
        pltpu.sync_copy(zeros_hbm.at[pl.ds(0, rem)],
                        acc.at[pl.ds(r0 + full * CH, rem)])

    @pl.when(s == NS - 1)
    def _():
        pltpu.sync_copy(zeros_hbm.at[pl.ds(0, RTAIL)],
                        acc.at[pl.ds(NS * RPT, RTAIL)])


def _flush_spmem(acc, out_hbm, c, s):
    r0 = s * RPT
    pltpu.sync_copy(acc.at[pl.ds(r0, RPT)], out_hbm.at[c, pl.ds(r0, RPT)])

    @pl.when(s == NS - 1)
    def _():
        pltpu.sync_copy(acc.at[pl.ds(NS * RPT, RTAIL)],
                        out_hbm.at[c, pl.ds(NS * RPT, RTAIL)])


@functools.partial(
    pl.kernel,
    out_type=jax.ShapeDtypeStruct((NC, N, D), jnp.float32),
    mesh=_mesh,
    scratch_types=[
        pltpu.VMEM((4, CH), jnp.int32),
        pltpu.VMEM((REM,), jnp.int32),
        pltpu.VMEM((CH, D), jnp.float32),
        pltpu.VMEM_SHARED((N, D), jnp.float32),
        [pltpu.SemaphoreType.DMA] * 4,
        [pltpu.SemaphoreType.DMA] * 2,
    ],
)
def _sc_degree(dst_hbm, ones_hbm, zeros_hbm, out_hbm, di, dr, ones_v, acc,
               dsem, ssem):
    c = lax.axis_index("c")
    s = lax.axis_index("s")
    ebase = (c * NS + s) * EPW

    _zero_spmem(zeros_hbm, acc, s)
    pltpu.sync_copy(ones_hbm, ones_v)
    plsc.subcore_barrier()

    def load_idx(jj, b):
        pltpu.async_copy(dst_hbm.at[pl.ds(ebase + jj * CH, CH)], di.at[b],
                         dsem[b])

    def wait_idx(b):
        pltpu.make_async_copy(dst_hbm.at[pl.ds(ebase, CH)], di.at[b],
                              dsem[b]).wait()

    def start_scatter(b, p):
        pltpu.async_copy(ones_v, acc.at[di.at[b]], ssem[p], add=True)

    def wait_scatter(b, p):
        pltpu.make_async_copy(ones_v, acc.at[di.at[b]], ssem[p]).wait()

    for b in range(4):
        load_idx(b, b)
    wait_idx(0)
    start_scatter(0, 0)
    wait_idx(1)
    start_scatter(1, 1)

    def body(i, carry):
        k0 = 2 + 4 * i
        for m in range(4):
            k, sk, sk2, p = k0 + m, (2 + m) % 4, m % 4, m % 2
            wait_scatter(sk2, p)
            load_idx(k + 2, sk2)
            wait_idx(sk)
            start_scatter(sk, p)
        return carry

    lax.fori_loop(0, (NFULL - 6) // 4, body, 0)

    for m in range(2):
        k, sk, sk2, p = NFULL - 4 + m, (2 + m) % 4, m % 4, m % 2
        wait_scatter(sk2, p)
        load_idx(k + 2, sk2)
        wait_idx(sk)
        start_scatter(sk, p)
    for m in range(2):
        sk, p = m % 4, m % 2
        wait_scatter((2 + m) % 4, p)
        wait_idx(sk)
        start_scatter(sk, p)
    wait_scatter(0, 0)
    wait_scatter(1, 1)

    eb = ebase + NFULL * CH
    pltpu.sync_copy(dst_hbm.at[pl.ds(eb, REM)], dr)
    pltpu.sync_copy(ones_v.at[pl.ds(0, REM)], acc.at[dr], add=True)
    plsc.subcore_barrier()

    _flush_spmem(acc, out_hbm, c, s)


@functools.partial(
    pl.kernel,
    out_type=jax.ShapeDtypeStruct((NC, N, D), jnp.float32),
    mesh=_mesh,
    scratch_types=[
        pltpu.VMEM((6, CHA), jnp.int32),
        pltpu.VMEM((6, CHA), jnp.int32),
        pltpu.VMEM((REMA,), jnp.int32),
        pltpu.VMEM((REMA,), jnp.int32),
        pltpu.VMEM((3, CHA, D), jnp.float32),
        pltpu.VMEM_SHARED((N, D), jnp.float32),
        [pltpu.SemaphoreType.DMA] * 3,
        [pltpu.SemaphoreType.DMA] * 6,
        [pltpu.SemaphoreType.DMA] * 3,
    ],
)
def _sc_edge_agg(hs_hbm, src_hbm, dst_hbm, zeros_hbm, out_hbm,
                 si, di, sr, dr, rows, acc, gsem, dsem, ssem):
    c = lax.axis_index("c")
    s = lax.axis_index("s")
    ebase = (c * NS + s) * EPW

    _zero_spmem(zeros_hbm, acc, s)
    plsc.subcore_barrier()

    def load_idx(jj, b):
        eb = ebase + jj * CHA
        pltpu.async_copy(src_hbm.at[pl.ds(eb, CHA)], si.at[b], dsem[b])
        pltpu.async_copy(dst_hbm.at[pl.ds(eb, CHA)], di.at[b], dsem[b])

    def wait_idx(b):
        pltpu.make_async_copy(src_hbm.at[pl.ds(ebase, CHA)], si.at[b],
                              dsem[b]).wait()
        pltpu.make_async_copy(dst_hbm.at[pl.ds(ebase, CHA)], di.at[b],
                              dsem[b]).wait()

    def start_gather(rb, ib):
        pltpu.async_copy(hs_hbm.at[si.at[ib]], rows.at[rb], gsem[rb])

    def wait_gather(rb):
        pltpu.make_async_copy(hs_hbm.at[si.at[0]], rows.at[rb],
                              gsem[rb]).wait()

    def start_scatter(rb, ib):
        pltpu.async_copy(rows.at[rb], acc.at[di.at[ib]], ssem[rb], add=True)

    def wait_scatter(rb):
        pltpu.make_async_copy(rows.at[rb], acc.at[di.at[0]],
                              ssem[rb]).wait()

    def step(k, rb, ib, prefetch_g, prefetch_i):
        wait_gather(rb)
        start_scatter(rb, ib)
        wait_scatter((rb + 2) % 3)
        if prefetch_g:
            wait_idx((ib + 2) % 6)
            start_gather((rb + 2) % 3, (ib + 2) % 6)
        if prefetch_i:
            load_idx(k + 3, (ib + 3) % 6)

    for b in range(4):
        load_idx(b, b)
    wait_idx(0)
    start_gather(0, 0)
    wait_idx(1)
    start_gather(1, 1)
    wait_gather(0)
    start_scatter(0, 0)
    wait_idx(2)
    start_gather(2, 2)
    step(1, 1, 1, True, True)
    step(2, 2, 2, True, True)

    NLOOP = (NFA - 3 - 8) // 6

    def body(i, carry):
        for m in range(6):
            step(3 + 6 * i + m, (3 + m) % 3, (3 + m) % 6, True, True)
        return carry

    lax.fori_loop(0, NLOOP, body, 0)

    for k in range(3 + 6 * NLOOP, NFA):
        step(k, k % 3, k % 6, k + 2 <= NFA - 1, k + 3 <= NFA - 1)
    wait_scatter((NFA - 1) % 3)

    eb = ebase + NFA * CHA
    pltpu.sync_copy(src_hbm.at[pl.ds(eb, REMA)], sr)
    pltpu.sync_copy(dst_hbm.at[pl.ds(eb, REMA)], dr)
    pltpu.async_copy(hs_hbm.at[sr], rows.at[0, pl.ds(0, REMA)],
                     gsem[0]).wait()
    pltpu.sync_copy(rows.at[0, pl.ds(0, REMA)], acc.at[dr], add=True)
    plsc.subcore_barrier()

    _flush_spmem(acc, out_hbm, c, s)


BT = 1000
GRID = N // BT


def _dinv_of(degp):
    deg = degp[0, :, 0] + degp[1, :, 0] + 1.0
    return lax.rsqrt(deg)[:, None]


def _tc1_body(x_ref, w_ref, degp_ref, o_ref):
    dinv = _dinv_of(degp_ref[...])
    o_ref[...] = jnp.dot(x_ref[...], w_ref[...],
                         preferred_element_type=jnp.float32) * dinv


def _tc2_body(p_ref, hs_ref, degp_ref, b_ref, w_ref, o_ref):
    dinv = _dinv_of(degp_ref[...])
    p = p_ref[...]
    a = jnp.maximum(dinv * (p[0] + p[1] + hs_ref[...]) + b_ref[...], 0.0)
    o_ref[...] = jnp.dot(a, w_ref[...],
                         preferred_element_type=jnp.float32) * dinv


def _tc3_body(p_ref, hs_ref, degp_ref, b_ref, o_ref):
    dinv = _dinv_of(degp_ref[...])
    p = p_ref[...]
    o_ref[...] = dinv * (p[0] + p[1] + hs_ref[...]) + b_ref[...]


_row_spec = pl.BlockSpec((BT, D), lambda i: (i, 0))
_p_spec = pl.BlockSpec((NC, BT, D), lambda i: (0, i, 0))
_degp_spec = pl.BlockSpec((NC, BT, D), lambda i: (0, i, 0))
_w_spec = pl.BlockSpec((D, D), lambda i: (0, 0))
_b_spec = pl.BlockSpec((1, D), lambda i: (0, 0))
_out_nd = jax.ShapeDtypeStruct((N, D), jnp.float32)

_tc1 = pl.pallas_call(
    _tc1_body, grid=(GRID,), out_shape=_out_nd,
    in_specs=[_row_spec, _w_spec, _degp_spec], out_specs=_row_spec)

_tc2 = pl.pallas_call(
    _tc2_body, grid=(GRID,), out_shape=_out_nd,
    in_specs=[_p_spec, _row_spec, _degp_spec, _b_spec, _w_spec],
    out_specs=_row_spec)

_tc3 = pl.pallas_call(
    _tc3_body, grid=(GRID,), out_shape=_out_nd,
    in_specs=[_p_spec, _row_spec, _degp_spec, _b_spec],
    out_specs=_row_spec)


def kernel(x, edge_index, W1, b1, W2, b2):
    src = edge_index[0]
    dst = edge_index[1]
    onesD = jnp.ones((CH, D), jnp.float32)
    zerosD = jnp.zeros((CH, D), jnp.float32)

    degp = _sc_degree(dst, onesD, zerosD)
    hs1 = _tc1(x, W1, degp)
    p1 = _sc_edge_agg(hs1, src, dst, zerosD)
    hs2 = _tc2(p1, hs1, degp, b1.reshape(1, D), W2)
    p2 = _sc_edge_agg(hs2, src, dst, zerosD)
    return _tc3(p2, hs2, degp, b2.reshape(1, D))
